# Initial kernel scaffold; baseline (speedup 1.0000x reference)
import functools

import jax
import jax.numpy as jnp
from jax import lax
from jax.experimental import pallas as pl
from jax.experimental.pallas import tpu as pltpu
from jax.experimental.pallas import tpu_sc as plsc

N = 10000
NP = 10240
E2 = 640000
CHUNK = 128
NTILES = 32
NSUB = 16
E_SUB = 20096
E_PAD = E_SUB * NTILES
N_CHUNKS = E_SUB // CHUNK
D0 = 148
DP0 = 160
D1 = 64
N_OPS = 120
N_GRAPHS = 16
R = 1280


def _make_sc_agg(dp):
    mesh = plsc.VectorSubcoreMesh(core_axis_name="c", subcore_axis_name="s")
    stripe = NP // NSUB

    @functools.partial(
        pl.kernel,
        out_type=jax.ShapeDtypeStruct((2, NP, dp), jnp.float32),
        mesh=mesh,
        scratch_types=[
            pltpu.VMEM((CHUNK,), jnp.int32),
            pltpu.VMEM((CHUNK,), jnp.int32),
            pltpu.VMEM((CHUNK, dp), jnp.float32),
            pltpu.VMEM_SHARED((NP, dp), jnp.float32),
            pltpu.SemaphoreType.DMA,
        ],
    )
    def sc_agg(x_hbm, src_hbm, dst_hbm, zeros_hbm, out_hbm,
               src_v, dst_v, rows_v, agg_sh, sem):
        c = lax.axis_index("c")
        s = lax.axis_index("s")
        pltpu.sync_copy(zeros_hbm, agg_sh.at[pl.ds(s * stripe, stripe)])
        plsc.subcore_barrier()
        base0 = (c * NSUB + s) * E_SUB

        def body(k, carry):
            base = base0 + k * CHUNK
            pltpu.sync_copy(src_hbm.at[pl.ds(base, CHUNK)], src_v)
            pltpu.sync_copy(dst_hbm.at[pl.ds(base, CHUNK)], dst_v)
            pltpu.async_copy(x_hbm.at[src_v], rows_v, sem).wait()
            pltpu.sync_copy(rows_v, agg_sh.at[dst_v], add=True)
            return carry

        lax.fori_loop(0, N_CHUNKS, body, 0)
        plsc.subcore_barrier()
        pltpu.sync_copy(agg_sh.at[pl.ds(s * stripe, stripe)],
                        out_hbm.at[c, pl.ds(s * stripe, stripe)])

    return sc_agg


_sc_agg_d0 = _make_sc_agg(DP0)
_sc_agg_d1 = _make_sc_agg(D1)


def _prep_body(nf_ref, opc_ref, emb_ref, out_ref):
    i = pl.program_id(0)
    nf = nf_ref[...]
    opc = opc_ref[...]
    oh = (opc == lax.broadcasted_iota(jnp.int32, (1, N_OPS), 1))
    embp = jnp.dot(oh.astype(jnp.float32), emb_ref[...],
                   preferred_element_type=jnp.float32)
    rid = i * R + lax.broadcasted_iota(jnp.int32, (R, 1), 0)
    ones = jnp.where(rid < N, 1.0, 0.0).astype(jnp.float32)
    pad = jnp.zeros((R, DP0 - D0 - 1), jnp.float32)
    out_ref[...] = jnp.concatenate([nf, embp, ones, pad], axis=1)


def _leaky(h):
    return jnp.where(h > 0, h, 0.01 * h)


def _sage_tail(agg_d, invd, x, WlT, bl, WrT, WlinT, blin, d):
    mean = agg_d * invd
    out = (jnp.dot(mean, WlT, preferred_element_type=jnp.float32) + bl
           + jnp.dot(x, WrT, preferred_element_type=jnp.float32))
    nrm = jnp.sqrt(jnp.sum(out * out, axis=1, keepdims=True))
    out = out / jnp.maximum(nrm, 1e-12)
    h = (jnp.dot(out, WlinT[:d], preferred_element_type=jnp.float32)
         + jnp.dot(x, WlinT[d:], preferred_element_type=jnp.float32) + blin)
    return _leaky(h)


def _layer0_body(p0_ref, p1_ref, xa_ref, WlT_ref, bl_ref, WrT_ref,
                 WlinT_ref, blin_ref, out_ref, invd_ref):
    agg = p0_ref[...] + p1_ref[...]
    xa = xa_ref[...]
    x = xa[:, :D0]
    deg = agg[:, D0:D0 + 1]
    invd = 1.0 / jnp.maximum(deg, 1.0)
    y = _sage_tail(agg[:, :D0], invd, x, WlT_ref[...], bl_ref[...],
                   WrT_ref[...], WlinT_ref[...], blin_ref[...], D0)
    i = pl.program_id(0)
    rid = i * R + lax.broadcasted_iota(jnp.int32, (R, 1), 0)
    valid = rid < N
    out_ref[...] = jnp.where(valid, y, 0.0)
    invd_ref[...] = jnp.where(valid, invd, 0.0)


def _layer_body(p0_ref, p1_ref, x_ref, invd_ref, WlT_ref, bl_ref, WrT_ref,
                WlinT_ref, blin_ref, out_ref):
    agg = p0_ref[...] + p1_ref[...]
    x = x_ref[...]
    y = _sage_tail(agg, invd_ref[...], x, WlT_ref[...], bl_ref[...],
                   WrT_ref[...], WlinT_ref[...], blin_ref[...], D1)
    i = pl.program_id(0)
    rid = i * R + lax.broadcasted_iota(jnp.int32, (R, 1), 0)
    out_ref[...] = jnp.where(rid < N, y, 0.0)


def _pool_body(x_ref, batch_ref, cfg_ref, Wp1T_ref, bp1_ref, Wp2T_ref,
               bp2_ref, out_ref):
    x = x_ref[...]
    b = batch_ref[...]
    gids = lax.broadcasted_iota(jnp.int32, (N_GRAPHS, 1), 0)
    oh = (gids == b).astype(jnp.float32)
    pooled = jnp.dot(oh, x, preferred_element_type=jnp.float32)
    z = jnp.concatenate([pooled, cfg_ref[...]], axis=1)
    h = _leaky(jnp.dot(z, Wp1T_ref[...], preferred_element_type=jnp.float32)
               + bp1_ref[...])
    out_ref[...] = (jnp.dot(h, Wp2T_ref[...],
                            preferred_element_type=jnp.float32)
                    + bp2_ref[...])


def _row_spec(d):
    return pl.BlockSpec((R, d), lambda i: (i, 0))


def _full_spec(shape):
    return pl.BlockSpec(shape, lambda i: tuple(0 for _ in shape))


def kernel(node_feat, node_opcode, edge_index, config_feat, n_configs, batch,
           op_emb,
           Wl0, bl0, Wr0, Wlin0, blin0,
           Wl1, bl1, Wr1, Wlin1, blin1,
           Wl2, bl2, Wr2, Wlin2, blin2,
           Wp1, bp1, Wp2, bp2):
    f32 = jnp.float32
    nf = jnp.pad(node_feat.astype(f32), ((0, NP - N), (0, 0)))
    opc = jnp.pad(node_opcode.astype(jnp.int32), (0, NP - N),
                  constant_values=N_OPS).reshape(NP, 1)
    ei = edge_index.astype(jnp.int32)
    src = jnp.pad(jnp.concatenate([ei[:, 0], ei[:, 1]]), (0, E_PAD - E2),
                  constant_values=NP - 1)
    dst = jnp.pad(jnp.concatenate([ei[:, 1], ei[:, 0]]), (0, E_PAD - E2),
                  constant_values=NP - 1)
    zeros0 = jnp.zeros((NP // NSUB, DP0), f32)
    zeros1 = jnp.zeros((NP // NSUB, D1), f32)

    nprog = NP // R
    x0 = pl.pallas_call(
        _prep_body,
        grid=(nprog,),
        in_specs=[_row_spec(140), pl.BlockSpec((R, 1), lambda i: (i, 0)),
                  _full_spec((N_OPS, 8))],
        out_specs=_row_spec(DP0),
        out_shape=jax.ShapeDtypeStruct((NP, DP0), f32),
    )(nf, opc, op_emb.astype(f32))

    def run_layer0(x0_, p, Wl, bl, Wr, Wlin, blin):
        return pl.pallas_call(
            _layer0_body,
            grid=(nprog,),
            in_specs=[_row_spec(DP0), _row_spec(DP0), _row_spec(DP0),
                      _full_spec((D0, D0)), _full_spec((1, D0)),
                      _full_spec((D0, D0)), _full_spec((2 * D0, D1)),
                      _full_spec((1, D1))],
            out_specs=[_row_spec(D1), pl.BlockSpec((R, 1), lambda i: (i, 0))],
            out_shape=[jax.ShapeDtypeStruct((NP, D1), f32),
                       jax.ShapeDtypeStruct((NP, 1), f32)],
        )(p[0], p[1], x0_, Wl.T, bl.reshape(1, -1), Wr.T, Wlin.T,
          blin.reshape(1, -1))

    def run_layer(x_, invd, p, Wl, bl, Wr, Wlin, blin):
        return pl.pallas_call(
            _layer_body,
            grid=(nprog,),
            in_specs=[_row_spec(D1), _row_spec(D1), _row_spec(D1),
                      pl.BlockSpec((R, 1), lambda i: (i, 0)),
                      _full_spec((D1, D1)), _full_spec((1, D1)),
                      _full_spec((D1, D1)), _full_spec((2 * D1, D1)),
                      _full_spec((1, D1))],
            out_specs=_row_spec(D1),
            out_shape=jax.ShapeDtypeStruct((NP, D1), f32),
        )(p[0], p[1], x_, invd, Wl.T, bl.reshape(1, -1), Wr.T, Wlin.T,
          blin.reshape(1, -1))

    p0 = _sc_agg_d0(x0, src, dst, zeros0)
    x1, invd = run_layer0(x0, p0, Wl0, bl0, Wr0, Wlin0, blin0)
    p1 = _sc_agg_d1(x1, src, dst, zeros1)
    x2 = run_layer(x1, invd, p1, Wl1, bl1, Wr1, Wlin1, blin1)
    p2 = _sc_agg_d1(x2, src, dst, zeros1)
    x3 = run_layer(x2, invd, p2, Wl2, bl2, Wr2, Wlin2, blin2)

    batch_pad = jnp.pad(batch.astype(jnp.int32), (0, NP - N),
                        constant_values=N_GRAPHS).reshape(1, NP)
    out = pl.pallas_call(
        _pool_body,
        grid=(1,),
        in_specs=[_full_spec((NP, D1)), _full_spec((1, NP)),
                  _full_spec((N_GRAPHS, 24)), _full_spec((D1 + 24, D1)),
                  _full_spec((1, D1)), _full_spec((D1, 1)),
                  _full_spec((1, 1))],
        out_specs=_full_spec((N_GRAPHS, 1)),
        out_shape=jax.ShapeDtypeStruct((N_GRAPHS, 1), f32),
    )(x3, batch_pad, config_feat.astype(f32), Wp1.T, bp1.reshape(1, -1),
      Wp2.T, bp2.reshape(1, -1))
    return out[:, 0]


# speedup vs baseline: 8.2548x; 8.2548x over previous
import functools

import jax
import jax.numpy as jnp
from jax import lax
from jax.experimental import pallas as pl
from jax.experimental.pallas import tpu as pltpu
from jax.experimental.pallas import tpu_sc as plsc

N = 10000
NP = 10240
E2 = 640000
CHUNK = 128
NTILES = 32
NSUB = 16
E_SUB = 20096
E_PAD = E_SUB * NTILES
N_CHUNKS = E_SUB // CHUNK
D0 = 148
DP0 = 160
D1 = 64
N_OPS = 120
N_GRAPHS = 16
R = 1280


@functools.lru_cache(maxsize=None)
def _make_sc_agg(dp):
    mesh = plsc.VectorSubcoreMesh(core_axis_name="c", subcore_axis_name="s",
                                  num_cores=2, num_subcores=NSUB)
    stripe = NP // NSUB

    @functools.partial(
        pl.kernel,
        out_type=jax.ShapeDtypeStruct((2, NP, dp), jnp.float32),
        mesh=mesh,
        scratch_types=[
            pltpu.VMEM((CHUNK,), jnp.int32),
            pltpu.VMEM((CHUNK,), jnp.int32),
            pltpu.VMEM((CHUNK, dp), jnp.float32),
            pltpu.VMEM_SHARED((NP, dp), jnp.float32),
            pltpu.SemaphoreType.DMA,
        ],
        compiler_params=pltpu.CompilerParams(use_tc_tiling_on_sc=False),
    )
    def sc_agg(x_hbm, src_hbm, dst_hbm, zeros_hbm, out_hbm,
               src_v, dst_v, rows_v, agg_sh, sem):
        c = lax.axis_index("c")
        s = lax.axis_index("s")
        pltpu.sync_copy(zeros_hbm, agg_sh.at[pl.ds(s * stripe, stripe)])
        plsc.subcore_barrier()
        base0 = (c * NSUB + s) * E_SUB

        def body(k, carry):
            base = base0 + k * CHUNK
            pltpu.sync_copy(src_hbm.at[pl.ds(base, CHUNK)], src_v)
            pltpu.sync_copy(dst_hbm.at[pl.ds(base, CHUNK)], dst_v)
            pltpu.async_copy(x_hbm.at[src_v], rows_v, sem).wait()
            pltpu.sync_copy(rows_v, agg_sh.at[dst_v], add=True)
            return carry

        lax.fori_loop(0, N_CHUNKS, body, 0)
        plsc.subcore_barrier()
        pltpu.sync_copy(agg_sh.at[pl.ds(s * stripe, stripe)],
                        out_hbm.at[c, pl.ds(s * stripe, stripe)])

    return sc_agg


def _prep_body(nf_ref, opc_ref, emb_ref, out_ref):
    i = pl.program_id(0)
    nf = nf_ref[...]
    opc = opc_ref[...]
    oh = (opc == lax.broadcasted_iota(jnp.int32, (1, N_OPS), 1))
    embp = jnp.dot(oh.astype(jnp.float32), emb_ref[...],
                   preferred_element_type=jnp.float32, precision=lax.Precision.HIGHEST)
    rid = i * R + lax.broadcasted_iota(jnp.int32, (R, 1), 0)
    ones = jnp.where(rid < N, 1.0, 0.0).astype(jnp.float32)
    pad = jnp.zeros((R, DP0 - D0 - 1), jnp.float32)
    out_ref[...] = jnp.concatenate([nf, embp, ones, pad], axis=1)


def _leaky(h):
    return jnp.where(h > 0, h, 0.01 * h)


def _sage_tail(agg_d, invd, x, WlT, bl, WrT, WlinT, blin, d):
    mean = agg_d * invd
    out = (jnp.dot(mean, WlT, preferred_element_type=jnp.float32, precision=lax.Precision.HIGHEST) + bl
           + jnp.dot(x, WrT, preferred_element_type=jnp.float32, precision=lax.Precision.HIGHEST))
    nrm = jnp.sqrt(jnp.sum(out * out, axis=1, keepdims=True))
    out = out / jnp.maximum(nrm, 1e-12)
    h = (jnp.dot(out, WlinT[:d], preferred_element_type=jnp.float32, precision=lax.Precision.HIGHEST)
         + jnp.dot(x, WlinT[d:], preferred_element_type=jnp.float32, precision=lax.Precision.HIGHEST) + blin)
    return _leaky(h)


def _layer0_body(p0_ref, p1_ref, xa_ref, WlT_ref, bl_ref, WrT_ref,
                 WlinT_ref, blin_ref, out_ref, invd_ref):
    agg = p0_ref[...] + p1_ref[...]
    xa = xa_ref[...]
    x = xa[:, :D0]
    deg = agg[:, D0:D0 + 1]
    invd = 1.0 / jnp.maximum(deg, 1.0)
    y = _sage_tail(agg[:, :D0], invd, x, WlT_ref[...], bl_ref[...],
                   WrT_ref[...], WlinT_ref[...], blin_ref[...], D0)
    i = pl.program_id(0)
    rid = i * R + lax.broadcasted_iota(jnp.int32, (R, 1), 0)
    valid = rid < N
    out_ref[...] = jnp.where(valid, y, 0.0)
    invd_ref[...] = jnp.where(valid, invd, 0.0)


def _layer_body(p0_ref, p1_ref, x_ref, invd_ref, WlT_ref, bl_ref, WrT_ref,
                WlinT_ref, blin_ref, out_ref):
    agg = p0_ref[...] + p1_ref[...]
    x = x_ref[...]
    y = _sage_tail(agg, invd_ref[...], x, WlT_ref[...], bl_ref[...],
                   WrT_ref[...], WlinT_ref[...], blin_ref[...], D1)
    i = pl.program_id(0)
    rid = i * R + lax.broadcasted_iota(jnp.int32, (R, 1), 0)
    out_ref[...] = jnp.where(rid < N, y, 0.0)


def _pool_body(x_ref, batch_ref, cfg_ref, Wp1T_ref, bp1_ref, Wp2T_ref,
               bp2_ref, out_ref):
    x = x_ref[...]
    b = batch_ref[...]
    gids = lax.broadcasted_iota(jnp.int32, (N_GRAPHS, 1), 0)
    oh = (gids == b).astype(jnp.float32)
    pooled = jnp.dot(oh, x, preferred_element_type=jnp.float32, precision=lax.Precision.HIGHEST)
    z = jnp.concatenate([pooled, cfg_ref[...]], axis=1)
    h = _leaky(jnp.dot(z, Wp1T_ref[...], preferred_element_type=jnp.float32, precision=lax.Precision.HIGHEST)
               + bp1_ref[...])
    out_ref[...] = (jnp.dot(h, Wp2T_ref[...],
                            preferred_element_type=jnp.float32, precision=lax.Precision.HIGHEST)
                    + bp2_ref[...])


def _row_spec(d):
    return pl.BlockSpec((R, d), lambda i: (i, 0))


def _full_spec(shape):
    return pl.BlockSpec(shape, lambda i: tuple(0 for _ in shape))


def kernel(node_feat, node_opcode, edge_index, config_feat, n_configs, batch,
           op_emb,
           Wl0, bl0, Wr0, Wlin0, blin0,
           Wl1, bl1, Wr1, Wlin1, blin1,
           Wl2, bl2, Wr2, Wlin2, blin2,
           Wp1, bp1, Wp2, bp2):
    f32 = jnp.float32
    nf = jnp.pad(node_feat.astype(f32), ((0, NP - N), (0, 0)))
    opc = jnp.pad(node_opcode.astype(jnp.int32), (0, NP - N),
                  constant_values=N_OPS).reshape(NP, 1)
    ei = edge_index.astype(jnp.int32)
    src = jnp.pad(jnp.concatenate([ei[:, 0], ei[:, 1]]), (0, E_PAD - E2),
                  constant_values=NP - 1)
    dst = jnp.pad(jnp.concatenate([ei[:, 1], ei[:, 0]]), (0, E_PAD - E2),
                  constant_values=NP - 1)
    zeros0 = jnp.zeros((NP // NSUB, DP0), f32)
    zeros1 = jnp.zeros((NP // NSUB, D1), f32)

    nprog = NP // R
    x0 = pl.pallas_call(
        _prep_body,
        grid=(nprog,),
        in_specs=[_row_spec(140), pl.BlockSpec((R, 1), lambda i: (i, 0)),
                  _full_spec((N_OPS, 8))],
        out_specs=_row_spec(DP0),
        out_shape=jax.ShapeDtypeStruct((NP, DP0), f32),
    )(nf, opc, op_emb.astype(f32))

    def run_layer0(x0_, p, Wl, bl, Wr, Wlin, blin):
        return pl.pallas_call(
            _layer0_body,
            grid=(nprog,),
            in_specs=[_row_spec(DP0), _row_spec(DP0), _row_spec(DP0),
                      _full_spec((D0, D0)), _full_spec((1, D0)),
                      _full_spec((D0, D0)), _full_spec((2 * D0, D1)),
                      _full_spec((1, D1))],
            out_specs=[_row_spec(D1), pl.BlockSpec((R, 1), lambda i: (i, 0))],
            out_shape=[jax.ShapeDtypeStruct((NP, D1), f32),
                       jax.ShapeDtypeStruct((NP, 1), f32)],
        )(p[0], p[1], x0_, Wl.T, bl.reshape(1, -1), Wr.T, Wlin.T,
          blin.reshape(1, -1))

    def run_layer(x_, invd, p, Wl, bl, Wr, Wlin, blin):
        return pl.pallas_call(
            _layer_body,
            grid=(nprog,),
            in_specs=[_row_spec(D1), _row_spec(D1), _row_spec(D1),
                      pl.BlockSpec((R, 1), lambda i: (i, 0)),
                      _full_spec((D1, D1)), _full_spec((1, D1)),
                      _full_spec((D1, D1)), _full_spec((2 * D1, D1)),
                      _full_spec((1, D1))],
            out_specs=_row_spec(D1),
            out_shape=jax.ShapeDtypeStruct((NP, D1), f32),
        )(p[0], p[1], x_, invd, Wl.T, bl.reshape(1, -1), Wr.T, Wlin.T,
          blin.reshape(1, -1))

    p0 = _make_sc_agg(DP0)(x0, src, dst, zeros0)
    x1, invd = run_layer0(x0, p0, Wl0, bl0, Wr0, Wlin0, blin0)
    p1 = _make_sc_agg(D1)(x1, src, dst, zeros1)
    x2 = run_layer(x1, invd, p1, Wl1, bl1, Wr1, Wlin1, blin1)
    p2 = _make_sc_agg(D1)(x2, src, dst, zeros1)
    x3 = run_layer(x2, invd, p2, Wl2, bl2, Wr2, Wlin2, blin2)

    batch_pad = jnp.pad(batch.astype(jnp.int32), (0, NP - N),
                        constant_values=N_GRAPHS).reshape(1, NP)
    out = pl.pallas_call(
        _pool_body,
        grid=(1,),
        in_specs=[_full_spec((NP, D1)), _full_spec((1, NP)),
                  _full_spec((N_GRAPHS, 24)), _full_spec((D1 + 24, D1)),
                  _full_spec((1, D1)), _full_spec((D1, 1)),
                  _full_spec((1, 1))],
        out_specs=_full_spec((N_GRAPHS, 1)),
        out_shape=jax.ShapeDtypeStruct((N_GRAPHS, 1), f32),
    )(x3, batch_pad, config_feat.astype(f32), Wp1.T, bp1.reshape(1, -1),
      Wp2.T, bp2.reshape(1, -1))
    return out[:, 0]


# speedup vs baseline: 8.9263x; 1.0813x over previous
import functools

import jax
import jax.numpy as jnp
from jax import lax
from jax.experimental import pallas as pl
from jax.experimental.pallas import tpu as pltpu
from jax.experimental.pallas import tpu_sc as plsc

N = 10000
NP = 10240
E2 = 640000
CHUNK = 128
NTILES = 32
NSUB = 16
N_CHUNKS = 316
E_SUB = N_CHUNKS * CHUNK
E_PAD = E_SUB * NSUB
D0 = 148
DP0 = 160
D1 = 64
N_OPS = 120
N_GRAPHS = 16
R = 1280


@functools.lru_cache(maxsize=None)
def _make_sc_agg(half):
    mesh = plsc.VectorSubcoreMesh(core_axis_name="c", subcore_axis_name="s",
                                  num_cores=2, num_subcores=NSUB)
    stripe = NP // NSUB

    @functools.partial(
        pl.kernel,
        out_type=jax.ShapeDtypeStruct((2, NP, half), jnp.float32),
        mesh=mesh,
        scratch_types=[
            pltpu.VMEM((CHUNK,), jnp.int32),
            pltpu.VMEM((CHUNK,), jnp.int32),
            pltpu.VMEM((CHUNK,), jnp.int32),
            pltpu.VMEM((CHUNK,), jnp.int32),
            pltpu.VMEM((CHUNK, half), jnp.float32),
            pltpu.VMEM((CHUNK, half), jnp.float32),
            pltpu.VMEM_SHARED((NP, half), jnp.float32),
            pltpu.SemaphoreType.DMA,
            pltpu.SemaphoreType.DMA,
        ],
        compiler_params=pltpu.CompilerParams(use_tc_tiling_on_sc=False),
    )
    def sc_agg(x_hbm, srcoff_hbm, dst_hbm, zeros_hbm, out_hbm,
               src_a, src_b, dst_a, dst_b, rows_a, rows_b, agg_sh,
               sem_a, sem_b):
        c = lax.axis_index("c")
        s = lax.axis_index("s")
        pltpu.sync_copy(zeros_hbm, agg_sh.at[pl.ds(s * stripe, stripe)])
        plsc.subcore_barrier()
        bs = c * E_PAD + s * E_SUB
        bd = s * E_SUB
        pltpu.sync_copy(srcoff_hbm.at[pl.ds(bs, CHUNK)], src_a)
        pltpu.async_copy(x_hbm.at[src_a], rows_a, sem_a)

        def body(i, carry):
            o0 = 2 * i * CHUNK
            o1 = o0 + CHUNK
            o2c = jnp.minimum(o0 + 2 * CHUNK, E_SUB - CHUNK)
            pltpu.sync_copy(srcoff_hbm.at[pl.ds(bs + o1, CHUNK)], src_b)
            cp_b = pltpu.async_copy(x_hbm.at[src_b], rows_b, sem_b)
            pltpu.sync_copy(dst_hbm.at[pl.ds(bd + o0, CHUNK)], dst_a)
            pltpu.make_async_copy(x_hbm.at[src_a], rows_a, sem_a).wait()
            pltpu.sync_copy(rows_a, agg_sh.at[dst_a], add=True)

            pltpu.sync_copy(srcoff_hbm.at[pl.ds(bs + o2c, CHUNK)], src_a)
            pltpu.async_copy(x_hbm.at[src_a], rows_a, sem_a)

            pltpu.sync_copy(dst_hbm.at[pl.ds(bd + o1, CHUNK)], dst_b)
            cp_b.wait()
            pltpu.sync_copy(rows_b, agg_sh.at[dst_b], add=True)
            return carry

        lax.fori_loop(0, N_CHUNKS // 2, body, 0)
        pltpu.make_async_copy(x_hbm.at[src_a], rows_a, sem_a).wait()
        plsc.subcore_barrier()
        pltpu.sync_copy(agg_sh.at[pl.ds(s * stripe, stripe)],
                        out_hbm.at[c, pl.ds(s * stripe, stripe)])

    return sc_agg


def _prep_body(nf_ref, opc_ref, emb_ref, out_ref):
    i = pl.program_id(0)
    nf = nf_ref[...]
    opc = opc_ref[...]
    oh = (opc == lax.broadcasted_iota(jnp.int32, (1, N_OPS), 1))
    embp = jnp.dot(oh.astype(jnp.float32), emb_ref[...],
                   preferred_element_type=jnp.float32, precision=lax.Precision.HIGHEST)
    rid = i * R + lax.broadcasted_iota(jnp.int32, (R, 1), 0)
    ones = jnp.where(rid < N, 1.0, 0.0).astype(jnp.float32)
    pad = jnp.zeros((R, DP0 - D0 - 1), jnp.float32)
    full = jnp.concatenate([nf, embp, ones, pad], axis=1)
    out_ref[0] = full[:, :DP0 // 2]
    out_ref[1] = full[:, DP0 // 2:]


def _leaky(h):
    return jnp.where(h > 0, h, 0.01 * h)


def _bdot(a, b):
    return jnp.dot(a.astype(jnp.bfloat16), b.astype(jnp.bfloat16),
                   preferred_element_type=jnp.float32)


def _sage_tail(agg_d, deg, x, WlT, bl, WrT, WlinT, blin, d):
    mean = agg_d / jnp.maximum(deg, 1.0)
    out = _bdot(mean, WlT) + bl + _bdot(x, WrT)
    nrm = jnp.sqrt(jnp.sum(out * out, axis=1, keepdims=True))
    out = out / jnp.maximum(nrm, 1e-12)
    h = _bdot(jnp.concatenate([out, x], axis=1), WlinT) + blin
    return _leaky(h)


def _layer0_body(pa_ref, pb_ref, xa_ref, xb_ref, WlT_ref, bl_ref, WrT_ref,
                 WlinT_ref, blin_ref, out_ref, deg_ref):
    agg = jnp.concatenate([pa_ref[0], pb_ref[0]], axis=1)
    x = jnp.concatenate([xa_ref[0], xb_ref[0]], axis=1)[:, :D0]
    deg = agg[:, D0:D0 + 1]
    y = _sage_tail(agg[:, :D0], deg, x, WlT_ref[...], bl_ref[...],
                   WrT_ref[...], WlinT_ref[...], blin_ref[...], D0)
    i = pl.program_id(0)
    rid = i * R + lax.broadcasted_iota(jnp.int32, (R, 1), 0)
    valid = rid < N
    y = jnp.where(valid, y, 0.0)
    out_ref[0] = y[:, :D1 // 2]
    out_ref[1] = y[:, D1 // 2:]
    deg_ref[...] = jnp.where(valid, deg, 0.0)


def _layer_mid_body(pa_ref, pb_ref, xa_ref, xb_ref, deg_ref, WlT_ref,
                    bl_ref, WrT_ref, WlinT_ref, blin_ref, out_ref):
    agg = jnp.concatenate([pa_ref[0], pb_ref[0]], axis=1)
    x = jnp.concatenate([xa_ref[0], xb_ref[0]], axis=1)
    y = _sage_tail(agg, deg_ref[...], x, WlT_ref[...], bl_ref[...],
                   WrT_ref[...], WlinT_ref[...], blin_ref[...], D1)
    i = pl.program_id(0)
    rid = i * R + lax.broadcasted_iota(jnp.int32, (R, 1), 0)
    y = jnp.where(rid < N, y, 0.0)
    out_ref[0] = y[:, :D1 // 2]
    out_ref[1] = y[:, D1 // 2:]


def _layer_last_body(pa_ref, pb_ref, xa_ref, xb_ref, deg_ref, WlT_ref,
                     bl_ref, WrT_ref, WlinT_ref, blin_ref, out_ref):
    agg = jnp.concatenate([pa_ref[0], pb_ref[0]], axis=1)
    x = jnp.concatenate([xa_ref[0], xb_ref[0]], axis=1)
    y = _sage_tail(agg, deg_ref[...], x, WlT_ref[...], bl_ref[...],
                   WrT_ref[...], WlinT_ref[...], blin_ref[...], D1)
    i = pl.program_id(0)
    rid = i * R + lax.broadcasted_iota(jnp.int32, (R, 1), 0)
    out_ref[...] = jnp.where(rid < N, y, 0.0)


def _pool_body(x_ref, batch_ref, cfg_ref, Wp1T_ref, bp1_ref, Wp2T_ref,
               bp2_ref, out_ref):
    x = x_ref[...]
    b = batch_ref[...]
    gids = lax.broadcasted_iota(jnp.int32, (N_GRAPHS, 1), 0)
    oh = (gids == b).astype(jnp.float32)
    pooled = jnp.dot(oh, x, preferred_element_type=jnp.float32,
                     precision=lax.Precision.HIGHEST)
    z = jnp.concatenate([pooled, cfg_ref[...]], axis=1)
    h = _leaky(_bdot(z, Wp1T_ref[...]) + bp1_ref[...])
    out_ref[...] = _bdot(h, Wp2T_ref[...]) + bp2_ref[...]


def _row_spec(d):
    return pl.BlockSpec((R, d), lambda i: (i, 0))


def _full_spec(shape):
    return pl.BlockSpec(shape, lambda i: tuple(0 for _ in shape))


def kernel(node_feat, node_opcode, edge_index, config_feat, n_configs, batch,
           op_emb,
           Wl0, bl0, Wr0, Wlin0, blin0,
           Wl1, bl1, Wr1, Wlin1, blin1,
           Wl2, bl2, Wr2, Wlin2, blin2,
           Wp1, bp1, Wp2, bp2):
    f32 = jnp.float32
    nf = jnp.pad(node_feat.astype(f32), ((0, NP - N), (0, 0)))
    opc = jnp.pad(node_opcode.astype(jnp.int32), (0, NP - N),
                  constant_values=N_OPS).reshape(NP, 1)
    ei = edge_index.astype(jnp.int32)
    src = jnp.pad(jnp.concatenate([ei[:, 0], ei[:, 1]]), (0, E_PAD - E2),
                  constant_values=NP - 1)
    srcoff = jnp.concatenate([src, src + NP])
    dst = jnp.pad(jnp.concatenate([ei[:, 1], ei[:, 0]]), (0, E_PAD - E2),
                  constant_values=NP - 1)
    H0 = DP0 // 2
    H1 = D1 // 2
    zeros0 = jnp.zeros((NP // NSUB, H0), f32)
    zeros1 = jnp.zeros((NP // NSUB, H1), f32)

    nprog = NP // R

    def _plane_spec(h, plane):
        return pl.BlockSpec((1, R, h), lambda i, p=plane: (p, i, 0))

    def _stack_spec(h):
        return pl.BlockSpec((2, R, h), lambda i: (0, i, 0))

    x0 = pl.pallas_call(
        _prep_body,
        grid=(nprog,),
        in_specs=[_row_spec(140), pl.BlockSpec((R, 1), lambda i: (i, 0)),
                  _full_spec((N_OPS, 8))],
        out_specs=_stack_spec(H0),
        out_shape=jax.ShapeDtypeStruct((2, NP, H0), f32),
    )(nf, opc, op_emb.astype(f32))

    def run_layer0(x0_, p, Wl, bl, Wr, Wlin, blin):
        return pl.pallas_call(
            _layer0_body,
            grid=(nprog,),
            in_specs=[_plane_spec(H0, 0), _plane_spec(H0, 1),
                      _plane_spec(H0, 0), _plane_spec(H0, 1),
                      _full_spec((D0, D0)), _full_spec((1, D0)),
                      _full_spec((D0, D0)), _full_spec((2 * D0, D1)),
                      _full_spec((1, D1))],
            out_specs=[_stack_spec(H1),
                       pl.BlockSpec((R, 1), lambda i: (i, 0))],
            out_shape=[jax.ShapeDtypeStruct((2, NP, H1), f32),
                       jax.ShapeDtypeStruct((NP, 1), f32)],
        )(p, p, x0_, x0_, Wl.T, bl.reshape(1, -1), Wr.T, Wlin.T,
          blin.reshape(1, -1))

    def run_layer(body, out_spec, out_shape, x_, invd, p,
                  Wl, bl, Wr, Wlin, blin):
        return pl.pallas_call(
            body,
            grid=(nprog,),
            in_specs=[_plane_spec(H1, 0), _plane_spec(H1, 1),
                      _plane_spec(H1, 0), _plane_spec(H1, 1),
                      pl.BlockSpec((R, 1), lambda i: (i, 0)),
                      _full_spec((D1, D1)), _full_spec((1, D1)),
                      _full_spec((D1, D1)), _full_spec((2 * D1, D1)),
                      _full_spec((1, D1))],
            out_specs=out_spec,
            out_shape=out_shape,
        )(p, p, x_, x_, invd, Wl.T, bl.reshape(1, -1), Wr.T, Wlin.T,
          blin.reshape(1, -1))

    p0 = _make_sc_agg(H0)(x0.reshape(2 * NP, H0), srcoff, dst, zeros0)
    x1, invd = run_layer0(x0, p0, Wl0, bl0, Wr0, Wlin0, blin0)
    p1 = _make_sc_agg(H1)(x1.reshape(2 * NP, H1), srcoff, dst, zeros1)
    x2 = run_layer(_layer_mid_body, _stack_spec(H1),
                   jax.ShapeDtypeStruct((2, NP, H1), f32),
                   x1, invd, p1, Wl1, bl1, Wr1, Wlin1, blin1)
    p2 = _make_sc_agg(H1)(x2.reshape(2 * NP, H1), srcoff, dst, zeros1)
    x3 = run_layer(_layer_last_body, _row_spec(D1),
                   jax.ShapeDtypeStruct((NP, D1), f32),
                   x2, invd, p2, Wl2, bl2, Wr2, Wlin2, blin2)

    batch_pad = jnp.pad(batch.astype(jnp.int32), (0, NP - N),
                        constant_values=N_GRAPHS).reshape(1, NP)
    out = pl.pallas_call(
        _pool_body,
        grid=(1,),
        in_specs=[_full_spec((NP, D1)), _full_spec((1, NP)),
                  _full_spec((N_GRAPHS, 24)), _full_spec((D1 + 24, D1)),
                  _full_spec((1, D1)), _full_spec((D1, 1)),
                  _full_spec((1, 1))],
        out_specs=_full_spec((N_GRAPHS, 1)),
        out_shape=jax.ShapeDtypeStruct((N_GRAPHS, 1), f32),
    )(x3, batch_pad, config_feat.astype(f32), Wp1.T, bp1.reshape(1, -1),
      Wp2.T, bp2.reshape(1, -1))
    return out[:, 0]
